# baseline (speedup 1.0000x reference)
import jax
import jax.numpy as jnp
from jax.experimental import pallas as pl

_B, _L, _C = 4096, 20, 1000
_N = _B * _L
_ROWS = 2048


def _onehot_block(x_ref, o_ref):
    xb = x_ref[...]
    act = xb[:, 0:1].astype(jnp.int32)
    dur = xb[:, 1:2]
    col = jax.lax.broadcasted_iota(jnp.int32, (_ROWS, 1024), 1)
    o_ref[...] = (col == act).astype(jnp.float32)
    o_ref[:, _C:_C + 1] = dur


def kernel(x):
    xf = x.reshape(_N, 2)
    out = pl.pallas_call(
        _onehot_block,
        grid=(_N // _ROWS,),
        in_specs=[pl.BlockSpec((_ROWS, 2), lambda i: (i, 0))],
        out_specs=pl.BlockSpec((_ROWS, 1024), lambda i: (i, 0)),
        out_shape=jax.ShapeDtypeStruct((_N, 1024), jnp.float32),
    )(xf)
    return out.reshape(_B, _L, 1024)


# speedup vs baseline: 1.0440x; 1.0440x over previous
import jax
import jax.numpy as jnp
from jax.experimental import pallas as pl
from jax.experimental.pallas import tpu as pltpu

_B, _L, _C = 4096, 20, 1000
_N = _B * _L
_ROWS = 1024
_NSTEP = _N // _ROWS
_NBUF = 8


def _onehot_multibuf(x_ref, o_ref, *scratch):
    bufs = scratch[:_NBUF]
    sems = scratch[_NBUF:]
    col = jax.lax.broadcasted_iota(jnp.int32, (_ROWS, _C + 1), 1)

    def step(go, carry):
        for b in range(_NBUF):
            i = go * _NBUF + b
            buf, sem = bufs[b], sems[b]

            @pl.when(go >= 1)
            def _wait_prev():
                pltpu.make_async_copy(
                    buf,
                    o_ref.at[pl.ds((i - _NBUF) * _ROWS, _ROWS), :],
                    sem,
                ).wait()

            xb = x_ref[:, pl.ds(i * _ROWS, _ROWS)]
            xt = jax.lax.transpose(xb, (1, 0))
            act = xt[:, 0:1].astype(jnp.int32)
            dur = xt[:, 1:2]
            buf[...] = (col == act).astype(jnp.float32)
            buf[:, _C:_C + 1] = dur
            pltpu.make_async_copy(
                buf,
                o_ref.at[pl.ds(i * _ROWS, _ROWS), :],
                sem,
            ).start()
        return carry

    jax.lax.fori_loop(0, _NSTEP // _NBUF, step, 0)

    for b in range(_NBUF):
        i = _NSTEP - _NBUF + b
        pltpu.make_async_copy(
            bufs[b],
            o_ref.at[pl.ds(i * _ROWS, _ROWS), :],
            sems[b],
        ).wait()


def kernel(x):
    xt = x.reshape(_N, 2).T
    out = pl.pallas_call(
        _onehot_multibuf,
        in_specs=[pl.BlockSpec(memory_space=pltpu.VMEM)],
        out_specs=pl.BlockSpec(memory_space=pl.ANY),
        out_shape=jax.ShapeDtypeStruct((_N, _C + 1), jnp.float32),
        scratch_shapes=(
            [pltpu.VMEM((_ROWS, _C + 1), jnp.float32) for _ in range(_NBUF)]
            + [pltpu.SemaphoreType.DMA for _ in range(_NBUF)]
        ),
    )(xt)
    return out.reshape(_B, _L, _C + 1)
